# baseline (speedup 1.0000x reference)
import dataclasses
import functools

import jax
import jax.numpy as jnp
from jax import lax
from jax.experimental import pallas as pl
from jax.experimental.pallas import tpu as pltpu
from jax.experimental.pallas import tpu_sc as plsc

N = 8192
E = 16384
B = 1024
H = 128
HE = 64
FN = 64
FE = 16
MR = 256
MAX_ATOMS = 24
NUM_STEPS = 8

def _mm(a, b):
    a_hi = a.astype(jnp.bfloat16)
    b_hi = b.astype(jnp.bfloat16)
    a_lo = (a - a_hi.astype(jnp.float32)).astype(jnp.bfloat16)
    b_lo = (b - b_hi.astype(jnp.float32)).astype(jnp.bfloat16)

    def bmm(x, y):
        return lax.dot_general(x, y, (((1,), (0,)), ((), ())),
                               preferred_element_type=jnp.float32)

    return bmm(a_hi, b_hi) + (bmm(a_hi, b_lo) + bmm(a_lo, b_hi))


def _node_init_body(nf_ref, wn_ref, bn_ref, wmh_ref, h_ref, hw_ref):
    h = jnp.maximum(_mm(nf_ref[...], wn_ref[...]) + bn_ref[...], 0.0)
    h_ref[...] = h
    hw_ref[...] = _mm(h, wmh_ref[...])


def _node_init(nf, Wn, bn, Wm_h):
    blk = 1024
    return pl.pallas_call(
        _node_init_body,
        grid=(N // blk,),
        in_specs=[
            pl.BlockSpec((blk, FN), lambda i: (i, 0)),
            pl.BlockSpec((FN, H), lambda i: (0, 0)),
            pl.BlockSpec((1, H), lambda i: (0, 0)),
            pl.BlockSpec((H, H), lambda i: (0, 0)),
        ],
        out_specs=[
            pl.BlockSpec((blk, H), lambda i: (i, 0)),
            pl.BlockSpec((blk, H), lambda i: (i, 0)),
        ],
        out_shape=[
            jax.ShapeDtypeStruct((N, H), jnp.float32),
            jax.ShapeDtypeStruct((N, H), jnp.float32),
        ],
    )(nf, Wn, bn, Wm_h)


def _edge_init_body(ef_ref, we_ref, be_ref, wme_ref, bm_ref, out_ref):
    eh = jnp.maximum(_mm(ef_ref[...], we_ref[...]) + be_ref[...], 0.0)
    out_ref[...] = _mm(eh, wme_ref[...]) + bm_ref[...]


def _edge_init(ef, We, be, Wm_e, bm):
    blk = 2048
    return pl.pallas_call(
        _edge_init_body,
        grid=(E // blk,),
        in_specs=[
            pl.BlockSpec((blk, FE), lambda i: (i, 0)),
            pl.BlockSpec((FE, HE), lambda i: (0, 0)),
            pl.BlockSpec((1, HE), lambda i: (0, 0)),
            pl.BlockSpec((HE, H), lambda i: (0, 0)),
            pl.BlockSpec((1, H), lambda i: (0, 0)),
        ],
        out_specs=pl.BlockSpec((blk, H), lambda i: (i, 0)),
        out_shape=jax.ShapeDtypeStruct((E, H), jnp.float32),
    )(ef, We, be, Wm_e, bm)


def _mol_init_body(mol_ref, w1m_ref, b1_ref, bi_ref, molw_ref, idx_ref):
    molw_ref[...] = _mm(mol_ref[...], w1m_ref[...]) + b1_ref[...]
    bcol = lax.broadcasted_iota(jnp.int32, (B, 1), 0)
    fi = jnp.zeros((B, 1), jnp.int32)
    cnt = jnp.zeros((B, 1), jnp.int32)
    for k in range(8):
        row = bi_ref[k, :].reshape(1, N // 8)
        fi = fi + jnp.sum((row < bcol).astype(jnp.int32), axis=1, keepdims=True)
        cnt = cnt + jnp.sum((row == bcol).astype(jnp.int32), axis=1, keepdims=True)
    j = lax.broadcasted_iota(jnp.int32, (B, MAX_ATOMS), 1)
    zrow = N + ((bcol * MAX_ATOMS + j) & 511)
    idx_ref[...] = jnp.where(j < cnt, fi + j, zrow)


def _mol_init(mol, W1_m, b1, bi2d):
    return pl.pallas_call(
        _mol_init_body,
        in_specs=[
            pl.BlockSpec((B, MR), lambda: (0, 0)),
            pl.BlockSpec((MR, 256), lambda: (0, 0)),
            pl.BlockSpec((1, 256), lambda: (0, 0)),
            pl.BlockSpec((8, N // 8), lambda: (0, 0)),
        ],
        out_specs=[
            pl.BlockSpec((B, 256), lambda: (0, 0)),
            pl.BlockSpec((B, MAX_ATOMS), lambda: (0, 0)),
        ],
        out_shape=[
            jax.ShapeDtypeStruct((B, 256), jnp.float32),
            jax.ShapeDtypeStruct((B, MAX_ATOMS), jnp.int32),
        ],
    )(mol, W1_m, b1, bi2d)


def _update_body(h_ref, agg_ref, wu_ref, bu_ref, wmh_ref,
                 h_out_ref, hw_out_ref):
    agg = agg_ref[0]
    for k in range(1, agg_ref.shape[0]):
        agg = agg + agg_ref[k]
    z = jnp.concatenate([h_ref[...], agg], axis=1)
    h_new = jnp.maximum(_mm(z, wu_ref[...]) + bu_ref[...], 0.0)
    h_out_ref[...] = h_new
    hw_out_ref[...] = _mm(h_new, wmh_ref[...])


def _update(h, aggs, Wu, bu, Wm_h):
    blk = 1024
    ka = aggs.shape[0]
    return pl.pallas_call(
        _update_body,
        grid=(N // blk,),
        in_specs=[
            pl.BlockSpec((blk, H), lambda i: (i, 0)),
            pl.BlockSpec((ka, blk, H), lambda i: (0, i, 0)),
            pl.BlockSpec((2 * H, H), lambda i: (0, 0)),
            pl.BlockSpec((1, H), lambda i: (0, 0)),
            pl.BlockSpec((H, H), lambda i: (0, 0)),
        ],
        out_specs=[
            pl.BlockSpec((blk, H), lambda i: (i, 0)),
            pl.BlockSpec((blk, H), lambda i: (i, 0)),
        ],
        out_shape=[
            jax.ShapeDtypeStruct((N, H), jnp.float32),
            jax.ShapeDtypeStruct((N, H), jnp.float32),
        ],
    )(h, aggs, Wu, bu, Wm_h)


_NW_EXT = N + 512


def _mlp_body(h_ref, agg_ref, wu_ref, bu_ref, nm_ref, w1h_ref,
              w2_ref, b2_ref, w3_ref, b3_ref, w4r_ref, b4_ref, out_ref):
    i = pl.program_id(0)
    agg = agg_ref[0]
    for k in range(1, agg_ref.shape[0]):
        agg = agg + agg_ref[k]
    z = jnp.concatenate([h_ref[...], agg], axis=1)
    h = jnp.maximum(_mm(z, wu_ref[...]) + bu_ref[...], 0.0)
    x1 = jnp.maximum(_mm(h, w1h_ref[...]) + nm_ref[...], 0.0)
    x2 = jnp.maximum(_mm(x1, w2_ref[...]) + b2_ref[...], 0.0)
    x3 = jnp.maximum(_mm(x2, w3_ref[...]) + b3_ref[...], 0.0)
    logit = jnp.sum(x3 * w4r_ref[...], axis=1, keepdims=True) + b4_ref[...]
    p = jax.nn.sigmoid(logit)
    w = h * p
    out_ref[...] = jnp.where(i >= N // 512, 0.0, w)


def _mlp(h, aggs, Wu, bu, nm, W1_h, W2, b2, W3, b3, W4r, b4):
    blk = 512
    last = N // blk - 1
    ka = aggs.shape[0]
    return pl.pallas_call(
        _mlp_body,
        grid=(_NW_EXT // blk,),
        in_specs=[
            pl.BlockSpec((blk, H), lambda i: (jnp.minimum(i, last), 0)),
            pl.BlockSpec((ka, blk, H), lambda i: (0, jnp.minimum(i, last), 0)),
            pl.BlockSpec((2 * H, H), lambda i: (0, 0)),
            pl.BlockSpec((1, H), lambda i: (0, 0)),
            pl.BlockSpec((blk, 256), lambda i: (jnp.minimum(i, last), 0)),
            pl.BlockSpec((H, 256), lambda i: (0, 0)),
            pl.BlockSpec((256, H), lambda i: (0, 0)),
            pl.BlockSpec((1, H), lambda i: (0, 0)),
            pl.BlockSpec((H, HE), lambda i: (0, 0)),
            pl.BlockSpec((1, HE), lambda i: (0, 0)),
            pl.BlockSpec((1, HE), lambda i: (0, 0)),
            pl.BlockSpec((1, 1), lambda i: (0, 0)),
        ],
        out_specs=pl.BlockSpec((blk, H), lambda i: (i, 0)),
        out_shape=jax.ShapeDtypeStruct((_NW_EXT, H), jnp.float32),
    )(h, aggs, Wu, bu, nm, W1_h, W2, b2, W3, b3, W4r, b4)


_NC = 2
_NS = 16
_NW = _NC * _NS

_sc_mesh = plsc.VectorSubcoreMesh(core_axis_name="c", subcore_axis_name="s")

_sc_params = pltpu.CompilerParams()
if "needs_layout_passes" in pltpu.CompilerParams.__dataclass_fields__:
    _sc_params = dataclasses.replace(_sc_params, needs_layout_passes=False)


def _sc_gather_rows(table, idx, nrows, d):
    idx2 = idx.reshape(1, nrows)

    @functools.partial(
        pl.kernel,
        out_type=jax.ShapeDtypeStruct((nrows, d), jnp.float32),
        mesh=_sc_mesh,
        compiler_params=_sc_params,
    )
    def k(x_hbm, i_hbm, o_hbm):
        def body(i_vmem, o_vmem):
            pltpu.sync_copy(x_hbm.at[i_vmem.at[0]], o_vmem)

        pltpu.emit_pipeline(
            body,
            grid=(nrows // 128,),
            in_specs=[pl.BlockSpec((1, 128), index_map=lambda i: (0, i))],
            out_specs=[pl.BlockSpec((128, d), index_map=lambda i: (i, 0))],
            core_axis_name=("c", "s"),
            dimension_semantics=(pltpu.PARALLEL,),
        )(i_hbm, o_hbm)

    return k(table, idx2)


def _sc_step(hW, ehp, src2d, dst2d):
    epw = E // _NW
    ch = 64
    nch = epw // ch
    rstride = N // _NS

    @functools.partial(
        pl.kernel,
        out_type=jax.ShapeDtypeStruct((_NC, N, H), jnp.float32),
        mesh=_sc_mesh,
        compiler_params=_sc_params,
        scratch_types=[
            pltpu.VMEM((nch, ch), jnp.int32),
            pltpu.VMEM((nch, ch), jnp.int32),
            pltpu.VMEM((2, ch, H), jnp.float32),
            pltpu.VMEM((2, ch, H), jnp.float32),
            pltpu.VMEM_SHARED((N, H), jnp.float32),
            pltpu.SemaphoreType.DMA,
            pltpu.SemaphoreType.DMA,
            pltpu.SemaphoreType.DMA,
            pltpu.SemaphoreType.DMA,
            pltpu.SemaphoreType.DMA,
            pltpu.SemaphoreType.DMA,
        ],
    )
    def k(hw_hbm, ehp_hbm, src_hbm, dst_hbm, out_hbm,
          sidx, didx, ehpb, rows, aggsh,
          gs0, gs1, es0, es1, ss0, ss1):
        cid = lax.axis_index("c")
        sid = lax.axis_index("s")
        wid = cid * _NS + sid
        gsem = (gs0, gs1)
        esem = (es0, es1)
        ssem = (ss0, ss1)

        @pl.loop(0, ch)
        def _(r):
            for l in range(H // 16):
                rows.at[0, r, pl.ds(l * 16, 16)][...] = jnp.zeros(
                    (16,), jnp.float32)

        for kk in range(rstride // ch):
            pltpu.sync_copy(rows.at[0],
                            aggsh.at[pl.ds(sid * rstride + kk * ch, ch), :])
        pltpu.sync_copy(src_hbm.at[pl.ds(wid * nch, nch), :], sidx)
        pltpu.sync_copy(dst_hbm.at[pl.ds(wid * nch, nch), :], didx)
        plsc.subcore_barrier()

        g = [None] * nch
        e = [None] * nch
        s = [None] * nch

        def issue(c):
            b = c % 2
            g[c] = pltpu.async_copy(hw_hbm.at[sidx.at[c]], rows.at[b],
                                    gsem[b])
            e[c] = pltpu.async_copy(
                ehp_hbm.at[pl.ds(wid * epw + c * ch, ch), :], ehpb.at[b],
                esem[b])

        issue(0)
        for c in range(nch):
            b = c % 2
            if c + 1 < nch:
                if c >= 1:
                    s[c - 1].wait()
                issue(c + 1)
            g[c].wait()
            e[c].wait()

            @pl.loop(0, ch)
            def _(r):
                for l in range(H // 16):
                    sl = pl.ds(l * 16, 16)
                    v = rows.at[b, r, sl][...] + ehpb.at[b, r, sl][...]
                    rows.at[b, r, sl][...] = jnp.maximum(v, 0.0)

            s[c] = pltpu.async_copy(rows.at[b], aggsh.at[didx.at[c]],
                                    ssem[b], add=True)

        s[nch - 2].wait()
        s[nch - 1].wait()
        plsc.subcore_barrier()
        pltpu.sync_copy(aggsh.at[pl.ds(sid * rstride, rstride), :],
                        out_hbm.at[cid, pl.ds(sid * rstride, rstride), :])

    return k(hW, ehp, src2d, dst2d)


def kernel(mol_reprs, node_features, edge_features, edges, batch_indices,
           Wn, bn, We, be, Wm, bm, Wu, bu, W1, b1, W2, b2, W3, b3, W4, b4):
    src = edges[0].astype(jnp.int32)
    dst = edges[1].astype(jnp.int32)
    bi = batch_indices.astype(jnp.int32)

    Wm_h, Wm_e = Wm[:H], Wm[H:]
    Wu_h, Wu_a = Wu[:H], Wu[H:]
    W1_h, W1_m = W1[:H], W1[H:]

    h, hW = _node_init(node_features, Wn, bn.reshape(1, H), Wm_h)
    ehp = _edge_init(edge_features, We, be.reshape(1, HE), Wm_e,
                     bm.reshape(1, H))
    molW, pidx = _mol_init(mol_reprs, W1_m, b1.reshape(1, 256),
                           bi.reshape(8, N // 8))

    src2d = src.reshape(E // 64, 64)
    dst2d = dst.reshape(E // 64, 64)
    for _ in range(NUM_STEPS - 1):
        aggs = _sc_step(hW, ehp, src2d, dst2d)
        h, hW = _update(h, aggs, Wu, bu.reshape(1, H), Wm_h)
    aggs = _sc_step(hW, ehp, src2d, dst2d)

    nm = _sc_gather_rows(molW, bi, N, 256)
    weighted = _mlp(h, aggs, Wu, bu.reshape(1, H), nm, W1_h, W2,
                    b2.reshape(1, H), W3, b3.reshape(1, HE),
                    W4.reshape(1, HE), b4.reshape(1, 1))

    out = _sc_gather_rows(weighted, pidx.reshape(B * MAX_ATOMS), B * MAX_ATOMS,
                          H)
    return out.reshape(B, MAX_ATOMS, H)


# speedup vs baseline: 1.0310x; 1.0310x over previous
import dataclasses
import functools

import jax
import jax.numpy as jnp
from jax import lax
from jax.experimental import pallas as pl
from jax.experimental.pallas import tpu as pltpu
from jax.experimental.pallas import tpu_sc as plsc

N = 8192
E = 16384
B = 1024
H = 128
HE = 64
FN = 64
FE = 16
MR = 256
MAX_ATOMS = 24
NUM_STEPS = 8

def _mm(a, b):
    a_hi = a.astype(jnp.bfloat16)
    b_hi = b.astype(jnp.bfloat16)
    a_lo = (a - a_hi.astype(jnp.float32)).astype(jnp.bfloat16)
    b_lo = (b - b_hi.astype(jnp.float32)).astype(jnp.bfloat16)

    def bmm(x, y):
        return lax.dot_general(x, y, (((1,), (0,)), ((), ())),
                               preferred_element_type=jnp.float32)

    return bmm(a_hi, b_hi) + (bmm(a_hi, b_lo) + bmm(a_lo, b_hi))


def _node_init_body(nf_ref, wn_ref, bn_ref, wmh_ref, h_ref, hw_ref):
    h = jnp.maximum(_mm(nf_ref[...], wn_ref[...]) + bn_ref[...], 0.0)
    h_ref[...] = h
    hw_ref[...] = _mm(h, wmh_ref[...])


def _node_init(nf, Wn, bn, Wm_h):
    blk = 1024
    return pl.pallas_call(
        _node_init_body,
        grid=(N // blk,),
        in_specs=[
            pl.BlockSpec((blk, FN), lambda i: (i, 0)),
            pl.BlockSpec((FN, H), lambda i: (0, 0)),
            pl.BlockSpec((1, H), lambda i: (0, 0)),
            pl.BlockSpec((H, H), lambda i: (0, 0)),
        ],
        out_specs=[
            pl.BlockSpec((blk, H), lambda i: (i, 0)),
            pl.BlockSpec((blk, H), lambda i: (i, 0)),
        ],
        out_shape=[
            jax.ShapeDtypeStruct((N, H), jnp.float32),
            jax.ShapeDtypeStruct((N, H), jnp.float32),
        ],
    )(nf, Wn, bn, Wm_h)


def _edge_init_body(ef_ref, we_ref, be_ref, wme_ref, bm_ref, out_ref):
    eh = jnp.maximum(_mm(ef_ref[...], we_ref[...]) + be_ref[...], 0.0)
    out_ref[...] = _mm(eh, wme_ref[...]) + bm_ref[...]


def _edge_init(ef, We, be, Wm_e, bm):
    blk = 2048
    return pl.pallas_call(
        _edge_init_body,
        grid=(E // blk,),
        in_specs=[
            pl.BlockSpec((blk, FE), lambda i: (i, 0)),
            pl.BlockSpec((FE, HE), lambda i: (0, 0)),
            pl.BlockSpec((1, HE), lambda i: (0, 0)),
            pl.BlockSpec((HE, H), lambda i: (0, 0)),
            pl.BlockSpec((1, H), lambda i: (0, 0)),
        ],
        out_specs=pl.BlockSpec((blk, H), lambda i: (i, 0)),
        out_shape=jax.ShapeDtypeStruct((E, H), jnp.float32),
    )(ef, We, be, Wm_e, bm)


def _mol_init_body(mol_ref, w1m_ref, b1_ref, bi_ref, molw_ref, idx_ref):
    molw_ref[...] = _mm(mol_ref[...], w1m_ref[...]) + b1_ref[...]
    bcol = lax.broadcasted_iota(jnp.int32, (B, 1), 0)
    fi = jnp.zeros((B, 1), jnp.int32)
    cnt = jnp.zeros((B, 1), jnp.int32)
    for k in range(8):
        row = bi_ref[k, :].reshape(1, N // 8)
        fi = fi + jnp.sum((row < bcol).astype(jnp.int32), axis=1, keepdims=True)
        cnt = cnt + jnp.sum((row == bcol).astype(jnp.int32), axis=1, keepdims=True)
    j = lax.broadcasted_iota(jnp.int32, (B, MAX_ATOMS), 1)
    zrow = N + ((bcol * MAX_ATOMS + j) & 511)
    idx_ref[...] = jnp.where(j < cnt, fi + j, zrow)


def _mol_init(mol, W1_m, b1, bi2d):
    return pl.pallas_call(
        _mol_init_body,
        in_specs=[
            pl.BlockSpec((B, MR), lambda: (0, 0)),
            pl.BlockSpec((MR, 256), lambda: (0, 0)),
            pl.BlockSpec((1, 256), lambda: (0, 0)),
            pl.BlockSpec((8, N // 8), lambda: (0, 0)),
        ],
        out_specs=[
            pl.BlockSpec((B, 256), lambda: (0, 0)),
            pl.BlockSpec((B, MAX_ATOMS), lambda: (0, 0)),
        ],
        out_shape=[
            jax.ShapeDtypeStruct((B, 256), jnp.float32),
            jax.ShapeDtypeStruct((B, MAX_ATOMS), jnp.int32),
        ],
    )(mol, W1_m, b1, bi2d)


def _update_body(h_ref, agg_ref, wu_ref, bu_ref, wmh_ref,
                 h_out_ref, hw_out_ref):
    agg = agg_ref[0]
    for k in range(1, agg_ref.shape[0]):
        agg = agg + agg_ref[k]
    z = jnp.concatenate([h_ref[...], agg], axis=1)
    h_new = jnp.maximum(_mm(z, wu_ref[...]) + bu_ref[...], 0.0)
    h_out_ref[...] = h_new
    hw_out_ref[...] = _mm(h_new, wmh_ref[...])


def _update(h, aggs, Wu, bu, Wm_h):
    blk = 1024
    ka = aggs.shape[0]
    return pl.pallas_call(
        _update_body,
        grid=(N // blk,),
        in_specs=[
            pl.BlockSpec((blk, H), lambda i: (i, 0)),
            pl.BlockSpec((ka, blk, H), lambda i: (0, i, 0)),
            pl.BlockSpec((2 * H, H), lambda i: (0, 0)),
            pl.BlockSpec((1, H), lambda i: (0, 0)),
            pl.BlockSpec((H, H), lambda i: (0, 0)),
        ],
        out_specs=[
            pl.BlockSpec((blk, H), lambda i: (i, 0)),
            pl.BlockSpec((blk, H), lambda i: (i, 0)),
        ],
        out_shape=[
            jax.ShapeDtypeStruct((N, H), jnp.float32),
            jax.ShapeDtypeStruct((N, H), jnp.float32),
        ],
    )(h, aggs, Wu, bu, Wm_h)


_NW_EXT = N + 512


def _mlp_body(h_ref, agg_ref, wu_ref, bu_ref, nm_ref, w1h_ref,
              w2_ref, b2_ref, w3_ref, b3_ref, w4r_ref, b4_ref, out_ref):
    i = pl.program_id(0)
    agg = agg_ref[0]
    for k in range(1, agg_ref.shape[0]):
        agg = agg + agg_ref[k]
    z = jnp.concatenate([h_ref[...], agg], axis=1)
    h = jnp.maximum(_mm(z, wu_ref[...]) + bu_ref[...], 0.0)
    x1 = jnp.maximum(_mm(h, w1h_ref[...]) + nm_ref[...], 0.0)
    x2 = jnp.maximum(_mm(x1, w2_ref[...]) + b2_ref[...], 0.0)
    x3 = jnp.maximum(_mm(x2, w3_ref[...]) + b3_ref[...], 0.0)
    logit = jnp.sum(x3 * w4r_ref[...], axis=1, keepdims=True) + b4_ref[...]
    p = jax.nn.sigmoid(logit)
    w = h * p
    out_ref[...] = jnp.where(i >= N // 512, 0.0, w)


def _mlp(h, aggs, Wu, bu, nm, W1_h, W2, b2, W3, b3, W4r, b4):
    blk = 512
    last = N // blk - 1
    ka = aggs.shape[0]
    return pl.pallas_call(
        _mlp_body,
        grid=(_NW_EXT // blk,),
        in_specs=[
            pl.BlockSpec((blk, H), lambda i: (jnp.minimum(i, last), 0)),
            pl.BlockSpec((ka, blk, H), lambda i: (0, jnp.minimum(i, last), 0)),
            pl.BlockSpec((2 * H, H), lambda i: (0, 0)),
            pl.BlockSpec((1, H), lambda i: (0, 0)),
            pl.BlockSpec((blk, 256), lambda i: (jnp.minimum(i, last), 0)),
            pl.BlockSpec((H, 256), lambda i: (0, 0)),
            pl.BlockSpec((256, H), lambda i: (0, 0)),
            pl.BlockSpec((1, H), lambda i: (0, 0)),
            pl.BlockSpec((H, HE), lambda i: (0, 0)),
            pl.BlockSpec((1, HE), lambda i: (0, 0)),
            pl.BlockSpec((1, HE), lambda i: (0, 0)),
            pl.BlockSpec((1, 1), lambda i: (0, 0)),
        ],
        out_specs=pl.BlockSpec((blk, H), lambda i: (i, 0)),
        out_shape=jax.ShapeDtypeStruct((_NW_EXT, H), jnp.float32),
    )(h, aggs, Wu, bu, nm, W1_h, W2, b2, W3, b3, W4r, b4)


_NC = 2
_NS = 16
_NW = _NC * _NS

_sc_mesh = plsc.VectorSubcoreMesh(core_axis_name="c", subcore_axis_name="s")

_sc_params = pltpu.CompilerParams()
if "needs_layout_passes" in pltpu.CompilerParams.__dataclass_fields__:
    _sc_params = dataclasses.replace(_sc_params, needs_layout_passes=False)


def _sc_gather_rows(table, idx, nrows, d):
    idx2 = idx.reshape(1, nrows)

    @functools.partial(
        pl.kernel,
        out_type=jax.ShapeDtypeStruct((nrows, d), jnp.float32),
        mesh=_sc_mesh,
        compiler_params=_sc_params,
    )
    def k(x_hbm, i_hbm, o_hbm):
        def body(i_vmem, o_vmem):
            pltpu.sync_copy(x_hbm.at[i_vmem.at[0]], o_vmem)

        pltpu.emit_pipeline(
            body,
            grid=(nrows // 128,),
            in_specs=[pl.BlockSpec((1, 128), index_map=lambda i: (0, i))],
            out_specs=[pl.BlockSpec((128, d), index_map=lambda i: (i, 0))],
            core_axis_name=("c", "s"),
            dimension_semantics=(pltpu.PARALLEL,),
        )(i_hbm, o_hbm)

    return k(table, idx2)


def _sc_step(hW, ehp, src2d, dst2d):
    epw = E // _NW
    ch = 64
    nch = epw // ch
    rstride = N // _NS

    @functools.partial(
        pl.kernel,
        out_type=jax.ShapeDtypeStruct((_NC, N, H), jnp.float32),
        mesh=_sc_mesh,
        compiler_params=_sc_params,
        scratch_types=[
            pltpu.VMEM((nch, ch), jnp.int32),
            pltpu.VMEM((nch, ch), jnp.int32),
            pltpu.VMEM((2, ch, H), jnp.float32),
            pltpu.VMEM((2, ch, H), jnp.float32),
            pltpu.VMEM_SHARED((N, H), jnp.float32),
            pltpu.SemaphoreType.DMA,
            pltpu.SemaphoreType.DMA,
            pltpu.SemaphoreType.DMA,
            pltpu.SemaphoreType.DMA,
            pltpu.SemaphoreType.DMA,
            pltpu.SemaphoreType.DMA,
        ],
    )
    def k(hw_hbm, ehp_hbm, src_hbm, dst_hbm, out_hbm,
          sidx, didx, ehpb, rows, aggsh,
          gs0, gs1, es0, es1, ss0, ss1):
        cid = lax.axis_index("c")
        sid = lax.axis_index("s")
        wid = cid * _NS + sid
        gsem = (gs0, gs1)
        esem = (es0, es1)
        ssem = (ss0, ss1)

        @pl.loop(0, ch)
        def _(r):
            for l in range(H // 16):
                rows.at[0, r, pl.ds(l * 16, 16)][...] = jnp.zeros(
                    (16,), jnp.float32)

        zcopies = [
            pltpu.async_copy(rows.at[0],
                             aggsh.at[pl.ds(sid * rstride + kk * ch, ch), :],
                             es0)
            for kk in range(rstride // ch)
        ]
        icopies = [
            pltpu.async_copy(src_hbm.at[pl.ds(wid * nch, nch), :], sidx, es1),
            pltpu.async_copy(dst_hbm.at[pl.ds(wid * nch, nch), :], didx, gs0),
        ]
        for cp in zcopies + icopies:
            cp.wait()
        plsc.subcore_barrier()

        g = [None] * nch
        e = [None] * nch
        s = [None] * nch

        def issue(c):
            b = c % 2
            g[c] = pltpu.async_copy(hw_hbm.at[sidx.at[c]], rows.at[b],
                                    gsem[b])
            e[c] = pltpu.async_copy(
                ehp_hbm.at[pl.ds(wid * epw + c * ch, ch), :], ehpb.at[b],
                esem[b])

        issue(0)
        for c in range(nch):
            b = c % 2
            if c + 1 < nch:
                if c >= 1:
                    s[c - 1].wait()
                issue(c + 1)
            g[c].wait()
            e[c].wait()

            @pl.loop(0, ch)
            def _(r):
                for l in range(H // 16):
                    sl = pl.ds(l * 16, 16)
                    v = rows.at[b, r, sl][...] + ehpb.at[b, r, sl][...]
                    rows.at[b, r, sl][...] = jnp.maximum(v, 0.0)

            s[c] = pltpu.async_copy(rows.at[b], aggsh.at[didx.at[c]],
                                    ssem[b], add=True)

        s[nch - 2].wait()
        s[nch - 1].wait()
        plsc.subcore_barrier()
        pltpu.sync_copy(aggsh.at[pl.ds(sid * rstride, rstride), :],
                        out_hbm.at[cid, pl.ds(sid * rstride, rstride), :])

    return k(hW, ehp, src2d, dst2d)


def kernel(mol_reprs, node_features, edge_features, edges, batch_indices,
           Wn, bn, We, be, Wm, bm, Wu, bu, W1, b1, W2, b2, W3, b3, W4, b4):
    src = edges[0].astype(jnp.int32)
    dst = edges[1].astype(jnp.int32)
    bi = batch_indices.astype(jnp.int32)

    Wm_h, Wm_e = Wm[:H], Wm[H:]
    Wu_h, Wu_a = Wu[:H], Wu[H:]
    W1_h, W1_m = W1[:H], W1[H:]

    h, hW = _node_init(node_features, Wn, bn.reshape(1, H), Wm_h)
    ehp = _edge_init(edge_features, We, be.reshape(1, HE), Wm_e,
                     bm.reshape(1, H))
    molW, pidx = _mol_init(mol_reprs, W1_m, b1.reshape(1, 256),
                           bi.reshape(8, N // 8))

    src2d = src.reshape(E // 64, 64)
    dst2d = dst.reshape(E // 64, 64)
    for _ in range(NUM_STEPS - 1):
        aggs = _sc_step(hW, ehp, src2d, dst2d)
        h, hW = _update(h, aggs, Wu, bu.reshape(1, H), Wm_h)
    aggs = _sc_step(hW, ehp, src2d, dst2d)

    nm = _sc_gather_rows(molW, bi, N, 256)
    weighted = _mlp(h, aggs, Wu, bu.reshape(1, H), nm, W1_h, W2,
                    b2.reshape(1, H), W3, b3.reshape(1, HE),
                    W4.reshape(1, HE), b4.reshape(1, 1))

    out = _sc_gather_rows(weighted, pidx.reshape(B * MAX_ATOMS), B * MAX_ATOMS,
                          H)
    return out.reshape(B, MAX_ATOMS, H)


# speedup vs baseline: 1.0608x; 1.0289x over previous
import dataclasses
import functools

import jax
import jax.numpy as jnp
from jax import lax
from jax.experimental import pallas as pl
from jax.experimental.pallas import tpu as pltpu
from jax.experimental.pallas import tpu_sc as plsc

N = 8192
E = 16384
B = 1024
H = 128
HE = 64
FN = 64
FE = 16
MR = 256
MAX_ATOMS = 24
NUM_STEPS = 8

def _mm(a, b):
    a_hi = a.astype(jnp.bfloat16)
    b_hi = b.astype(jnp.bfloat16)
    a_lo = (a - a_hi.astype(jnp.float32)).astype(jnp.bfloat16)
    b_lo = (b - b_hi.astype(jnp.float32)).astype(jnp.bfloat16)

    def bmm(x, y):
        return lax.dot_general(x, y, (((1,), (0,)), ((), ())),
                               preferred_element_type=jnp.float32)

    return bmm(a_hi, b_hi) + (bmm(a_hi, b_lo) + bmm(a_lo, b_hi))


def _node_init_body(nf_ref, wn_ref, bn_ref, wmh_ref, h_ref, hw_ref):
    h = jnp.maximum(_mm(nf_ref[...], wn_ref[...]) + bn_ref[...], 0.0)
    h_ref[...] = h
    hw_ref[...] = _mm(h, wmh_ref[...])


def _node_init(nf, Wn, bn, Wm_h):
    blk = 1024
    return pl.pallas_call(
        _node_init_body,
        grid=(N // blk,),
        in_specs=[
            pl.BlockSpec((blk, FN), lambda i: (i, 0)),
            pl.BlockSpec((FN, H), lambda i: (0, 0)),
            pl.BlockSpec((1, H), lambda i: (0, 0)),
            pl.BlockSpec((H, H), lambda i: (0, 0)),
        ],
        out_specs=[
            pl.BlockSpec((blk, H), lambda i: (i, 0)),
            pl.BlockSpec((blk, H), lambda i: (i, 0)),
        ],
        out_shape=[
            jax.ShapeDtypeStruct((N, H), jnp.float32),
            jax.ShapeDtypeStruct((N, H), jnp.float32),
        ],
    )(nf, Wn, bn, Wm_h)


def _edge_init_body(ef_ref, we_ref, be_ref, wme_ref, bm_ref, out_ref):
    eh = jnp.maximum(_mm(ef_ref[...], we_ref[...]) + be_ref[...], 0.0)
    out_ref[...] = _mm(eh, wme_ref[...]) + bm_ref[...]


def _edge_init(ef, We, be, Wm_e, bm):
    blk = 2048
    return pl.pallas_call(
        _edge_init_body,
        grid=(E // blk,),
        in_specs=[
            pl.BlockSpec((blk, FE), lambda i: (i, 0)),
            pl.BlockSpec((FE, HE), lambda i: (0, 0)),
            pl.BlockSpec((1, HE), lambda i: (0, 0)),
            pl.BlockSpec((HE, H), lambda i: (0, 0)),
            pl.BlockSpec((1, H), lambda i: (0, 0)),
        ],
        out_specs=pl.BlockSpec((blk, H), lambda i: (i, 0)),
        out_shape=jax.ShapeDtypeStruct((E, H), jnp.float32),
    )(ef, We, be, Wm_e, bm)


def _mol_init_body(mol_ref, w1m_ref, b1_ref, bi_ref, molw_ref, idx_ref):
    molw_ref[...] = _mm(mol_ref[...], w1m_ref[...]) + b1_ref[...]
    bcol = lax.broadcasted_iota(jnp.int32, (B, 1), 0)
    fi = jnp.zeros((B, 1), jnp.int32)
    cnt = jnp.zeros((B, 1), jnp.int32)
    for k in range(8):
        row = bi_ref[k, :].reshape(1, N // 8)
        fi = fi + jnp.sum((row < bcol).astype(jnp.int32), axis=1, keepdims=True)
        cnt = cnt + jnp.sum((row == bcol).astype(jnp.int32), axis=1, keepdims=True)
    j = lax.broadcasted_iota(jnp.int32, (B, MAX_ATOMS), 1)
    zrow = N + ((bcol * MAX_ATOMS + j) & 511)
    idx_ref[...] = jnp.where(j < cnt, fi + j, zrow)


def _mol_init(mol, W1_m, b1, bi2d):
    return pl.pallas_call(
        _mol_init_body,
        in_specs=[
            pl.BlockSpec((B, MR), lambda: (0, 0)),
            pl.BlockSpec((MR, 256), lambda: (0, 0)),
            pl.BlockSpec((1, 256), lambda: (0, 0)),
            pl.BlockSpec((8, N // 8), lambda: (0, 0)),
        ],
        out_specs=[
            pl.BlockSpec((B, 256), lambda: (0, 0)),
            pl.BlockSpec((B, MAX_ATOMS), lambda: (0, 0)),
        ],
        out_shape=[
            jax.ShapeDtypeStruct((B, 256), jnp.float32),
            jax.ShapeDtypeStruct((B, MAX_ATOMS), jnp.int32),
        ],
    )(mol, W1_m, b1, bi2d)


def _update_body(h_ref, agg_ref, wu_ref, bu_ref, wmh_ref,
                 h_out_ref, hw_out_ref):
    agg = agg_ref[0]
    for k in range(1, agg_ref.shape[0]):
        agg = agg + agg_ref[k]
    z = jnp.concatenate([h_ref[...], agg], axis=1)
    h_new = jnp.maximum(_mm(z, wu_ref[...]) + bu_ref[...], 0.0)
    h_out_ref[...] = h_new
    hw_out_ref[...] = _mm(h_new, wmh_ref[...])


def _update(h, aggs, Wu, bu, Wm_h):
    blk = 1024
    ka = aggs.shape[0]
    return pl.pallas_call(
        _update_body,
        grid=(N // blk,),
        in_specs=[
            pl.BlockSpec((blk, H), lambda i: (i, 0)),
            pl.BlockSpec((ka, blk, H), lambda i: (0, i, 0)),
            pl.BlockSpec((2 * H, H), lambda i: (0, 0)),
            pl.BlockSpec((1, H), lambda i: (0, 0)),
            pl.BlockSpec((H, H), lambda i: (0, 0)),
        ],
        out_specs=[
            pl.BlockSpec((blk, H), lambda i: (i, 0)),
            pl.BlockSpec((blk, H), lambda i: (i, 0)),
        ],
        out_shape=[
            jax.ShapeDtypeStruct((N, H), jnp.float32),
            jax.ShapeDtypeStruct((N, H), jnp.float32),
        ],
    )(h, aggs, Wu, bu, Wm_h)


_NW_EXT = N + 512


def _mlp_body(h_ref, agg_ref, wu_ref, bu_ref, nm_ref, w1h_ref,
              w2_ref, b2_ref, w3_ref, b3_ref, w4r_ref, b4_ref, out_ref):
    i = pl.program_id(0)
    agg = agg_ref[0]
    for k in range(1, agg_ref.shape[0]):
        agg = agg + agg_ref[k]
    z = jnp.concatenate([h_ref[...], agg], axis=1)
    h = jnp.maximum(_mm(z, wu_ref[...]) + bu_ref[...], 0.0)
    x1 = jnp.maximum(_mm(h, w1h_ref[...]) + nm_ref[...], 0.0)
    x2 = jnp.maximum(_mm(x1, w2_ref[...]) + b2_ref[...], 0.0)
    x3 = jnp.maximum(_mm(x2, w3_ref[...]) + b3_ref[...], 0.0)
    logit = jnp.sum(x3 * w4r_ref[...], axis=1, keepdims=True) + b4_ref[...]
    p = jax.nn.sigmoid(logit)
    w = h * p
    out_ref[...] = jnp.where(i >= N // 512, 0.0, w)


def _mlp(h, aggs, Wu, bu, nm, W1_h, W2, b2, W3, b3, W4r, b4):
    blk = 512
    last = N // blk - 1
    ka = aggs.shape[0]
    return pl.pallas_call(
        _mlp_body,
        grid=(_NW_EXT // blk,),
        in_specs=[
            pl.BlockSpec((blk, H), lambda i: (jnp.minimum(i, last), 0)),
            pl.BlockSpec((ka, blk, H), lambda i: (0, jnp.minimum(i, last), 0)),
            pl.BlockSpec((2 * H, H), lambda i: (0, 0)),
            pl.BlockSpec((1, H), lambda i: (0, 0)),
            pl.BlockSpec((blk, 256), lambda i: (jnp.minimum(i, last), 0)),
            pl.BlockSpec((H, 256), lambda i: (0, 0)),
            pl.BlockSpec((256, H), lambda i: (0, 0)),
            pl.BlockSpec((1, H), lambda i: (0, 0)),
            pl.BlockSpec((H, HE), lambda i: (0, 0)),
            pl.BlockSpec((1, HE), lambda i: (0, 0)),
            pl.BlockSpec((1, HE), lambda i: (0, 0)),
            pl.BlockSpec((1, 1), lambda i: (0, 0)),
        ],
        out_specs=pl.BlockSpec((blk, H), lambda i: (i, 0)),
        out_shape=jax.ShapeDtypeStruct((_NW_EXT, H), jnp.float32),
    )(h, aggs, Wu, bu, nm, W1_h, W2, b2, W3, b3, W4r, b4)


_NC = 2
_NS = 16
_NW = _NC * _NS

_sc_mesh = plsc.VectorSubcoreMesh(core_axis_name="c", subcore_axis_name="s")

_sc_params = pltpu.CompilerParams()
if "needs_layout_passes" in pltpu.CompilerParams.__dataclass_fields__:
    _sc_params = dataclasses.replace(_sc_params, needs_layout_passes=False)


def _sc_gather_rows(table, idx, nrows, d):
    idx2 = idx.reshape(1, nrows)

    @functools.partial(
        pl.kernel,
        out_type=jax.ShapeDtypeStruct((nrows, d), jnp.float32),
        mesh=_sc_mesh,
        compiler_params=_sc_params,
    )
    def k(x_hbm, i_hbm, o_hbm):
        def body(i_vmem, o_vmem):
            pltpu.sync_copy(x_hbm.at[i_vmem.at[0]], o_vmem)

        pltpu.emit_pipeline(
            body,
            grid=(nrows // 128,),
            in_specs=[pl.BlockSpec((1, 128), index_map=lambda i: (0, i))],
            out_specs=[pl.BlockSpec((128, d), index_map=lambda i: (i, 0))],
            core_axis_name=("c", "s"),
            dimension_semantics=(pltpu.PARALLEL,),
        )(i_hbm, o_hbm)

    return k(table, idx2)


def _sc_step(hW, ehp, src2d, dst2d):
    epw = E // _NW
    ch = 64
    nch = epw // ch
    rstride = N // _NS

    @functools.partial(
        pl.kernel,
        out_type=jax.ShapeDtypeStruct((_NC, N, H), jnp.float32),
        mesh=_sc_mesh,
        compiler_params=_sc_params,
        scratch_types=[
            pltpu.VMEM((nch, ch), jnp.int32),
            pltpu.VMEM((nch, ch), jnp.int32),
            pltpu.VMEM((2, ch, H), jnp.float32),
            pltpu.VMEM((2, ch, H), jnp.float32),
            pltpu.VMEM((ch, H), jnp.float32),
            pltpu.VMEM_SHARED((N, H), jnp.float32),
            pltpu.SemaphoreType.DMA,
            pltpu.SemaphoreType.DMA,
            pltpu.SemaphoreType.DMA,
            pltpu.SemaphoreType.DMA,
            pltpu.SemaphoreType.DMA,
            pltpu.SemaphoreType.DMA,
        ],
    )
    def k(hw_hbm, ehp_hbm, src_hbm, dst_hbm, out_hbm,
          sidx, didx, ehpb, rows, zbuf, aggsh,
          gs0, gs1, es0, es1, ss0, ss1):
        cid = lax.axis_index("c")
        sid = lax.axis_index("s")
        wid = cid * _NS + sid
        gsem = (gs0, gs1)
        esem = (es0, es1)
        ssem = (ss0, ss1)

        @pl.loop(0, ch)
        def _(r):
            for l in range(H // 16):
                zbuf.at[r, pl.ds(l * 16, 16)][...] = jnp.zeros(
                    (16,), jnp.float32)

        zcopies = [
            pltpu.async_copy(zbuf,
                             aggsh.at[pl.ds(sid * rstride + kk * ch, ch), :],
                             ss0)
            for kk in range(rstride // ch)
        ]
        icopies = [
            pltpu.async_copy(src_hbm.at[pl.ds(wid * nch, nch), :], sidx, es1),
            pltpu.async_copy(dst_hbm.at[pl.ds(wid * nch, nch), :], didx, gs0),
        ]
        for cp in icopies:
            cp.wait()

        g = [None] * nch
        e = [None] * nch
        s = [None] * nch

        def issue(c):
            b = c % 2
            g[c] = pltpu.async_copy(hw_hbm.at[sidx.at[c]], rows.at[b],
                                    gsem[b])
            e[c] = pltpu.async_copy(
                ehp_hbm.at[pl.ds(wid * epw + c * ch, ch), :], ehpb.at[b],
                esem[b])

        issue(0)
        for c in range(nch):
            b = c % 2
            if c + 1 < nch:
                if c >= 1:
                    s[c - 1].wait()
                issue(c + 1)
            g[c].wait()
            e[c].wait()

            @pl.loop(0, ch)
            def _(r):
                for l in range(H // 16):
                    sl = pl.ds(l * 16, 16)
                    v = rows.at[b, r, sl][...] + ehpb.at[b, r, sl][...]
                    rows.at[b, r, sl][...] = jnp.maximum(v, 0.0)

            if c == 0:
                for cp in zcopies:
                    cp.wait()
                plsc.subcore_barrier()
            s[c] = pltpu.async_copy(rows.at[b], aggsh.at[didx.at[c]],
                                    ssem[b], add=True)

        s[nch - 2].wait()
        s[nch - 1].wait()
        plsc.subcore_barrier()
        pltpu.sync_copy(aggsh.at[pl.ds(sid * rstride, rstride), :],
                        out_hbm.at[cid, pl.ds(sid * rstride, rstride), :])

    return k(hW, ehp, src2d, dst2d)


def kernel(mol_reprs, node_features, edge_features, edges, batch_indices,
           Wn, bn, We, be, Wm, bm, Wu, bu, W1, b1, W2, b2, W3, b3, W4, b4):
    src = edges[0].astype(jnp.int32)
    dst = edges[1].astype(jnp.int32)
    bi = batch_indices.astype(jnp.int32)

    Wm_h, Wm_e = Wm[:H], Wm[H:]
    Wu_h, Wu_a = Wu[:H], Wu[H:]
    W1_h, W1_m = W1[:H], W1[H:]

    h, hW = _node_init(node_features, Wn, bn.reshape(1, H), Wm_h)
    ehp = _edge_init(edge_features, We, be.reshape(1, HE), Wm_e,
                     bm.reshape(1, H))
    molW, pidx = _mol_init(mol_reprs, W1_m, b1.reshape(1, 256),
                           bi.reshape(8, N // 8))

    src2d = src.reshape(E // 64, 64)
    dst2d = dst.reshape(E // 64, 64)
    for _ in range(NUM_STEPS - 1):
        aggs = _sc_step(hW, ehp, src2d, dst2d)
        h, hW = _update(h, aggs, Wu, bu.reshape(1, H), Wm_h)
    aggs = _sc_step(hW, ehp, src2d, dst2d)

    nm = _sc_gather_rows(molW, bi, N, 256)
    weighted = _mlp(h, aggs, Wu, bu.reshape(1, H), nm, W1_h, W2,
                    b2.reshape(1, H), W3, b3.reshape(1, HE),
                    W4.reshape(1, HE), b4.reshape(1, 1))

    out = _sc_gather_rows(weighted, pidx.reshape(B * MAX_ATOMS), B * MAX_ATOMS,
                          H)
    return out.reshape(B, MAX_ATOMS, H)
